# baseline (speedup 1.0000x reference)
import functools

import jax
import jax.numpy as jnp
import numpy as np
from jax import lax
from jax.experimental import pallas as pl
from jax.experimental.pallas import tpu as pltpu
from jax.experimental.pallas import tpu_sc as plsc

N = 10000
E = 320000
D_IN = 128
HGC = 128
LG = 4
NUM_CLASSES = 2
EDGENET_DIM = 16
PAE_IN = EDGENET_DIM // 2
BN_EPS = 1e-5

NC = 2
NS = 16
NW = NC * NS
CHUNK = 128
CPW = 80
EP = NW * CPW * CHUNK
EPW = CPW * CHUNK
NPAD = 10240
ZROWS = NPAD // NS // 5

_mesh = plsc.VectorSubcoreMesh(
    core_axis_name="c", subcore_axis_name="s", num_cores=NC, num_subcores=NS)


IB = 16
NPW = NPAD // NW
EP1 = EP + IB * CHUNK


def _prop_body(x_hbm, row_hbm, col_hbm, ew_hbm, bnd_hbm, out_hbm,
               rowv, colv, ewv, gbuf0, gbuf1, accum, bv, gsem0, gsem1):
    cid = lax.axis_index("c")
    sid = lax.axis_index("s")
    wid = cid * NS + sid
    colbase = wid * NPW

    zero = jnp.zeros((16,), jnp.float32)
    def _zrow(i, _):
        for k in range(8):
            accum[i, pl.ds(k * 16, 16)] = zero
        return 0
    lax.fori_loop(0, NPW, _zrow, 0)

    pltpu.sync_copy(bnd_hbm.at[pl.ds(wid * 16, 16)], bv)
    bvv = bv[pl.ds(0, 16)]
    start = bvv[0]
    end = bvv[1]
    c0 = lax.shift_right_logical(start, 7)
    cend = lax.shift_right_logical(end + 127, 7)
    nb = lax.shift_right_logical(cend - c0 + 15, 4)
    iota16 = lax.iota(jnp.int32, 16)

    def _process(buf, jj, cb):
        eg_base = (cb + jj) * CHUNK

        def _group(g, _):
            sl = pl.ds(jj * CHUNK + g * 16, 16)
            rv = rowv[sl]
            cv = colv[sl]
            ev = ewv[sl]
            eg = eg_base + g * 16 + iota16
            inr = (eg >= start) & (eg < end) & (rv != cv)
            sv = jnp.where(inr, ev, 0.0)
            for e in range(16):
                s = sv[e]
                off = jnp.clip(cv[e] - colbase, 0, NPW - 1)
                for k in range(8):
                    ksl = pl.ds(k * 16, 16)
                    accum[off, ksl] = accum[off, ksl] + buf[g * 16 + e, ksl] * s
            return 0
        lax.fori_loop(0, CHUNK // 16, _group, 0)

    def _block(b, _):
        cb = c0 + b * IB
        pltpu.sync_copy(row_hbm.at[pl.ds(cb * CHUNK, IB * CHUNK)], rowv)
        pltpu.sync_copy(col_hbm.at[pl.ds(cb * CHUNK, IB * CHUNK)], colv)
        pltpu.sync_copy(ew_hbm.at[pl.ds(cb * CHUNK, IB * CHUNK)], ewv)
        pltpu.async_copy(x_hbm.at[rowv.at[pl.ds(0, CHUNK)]], gbuf0, gsem0)
        pltpu.async_copy(x_hbm.at[rowv.at[pl.ds(CHUNK, CHUNK)]], gbuf1, gsem1)

        def _pair(q, _):
            j0 = 2 * q
            j1 = j0 + 1
            pltpu.make_async_copy(
                x_hbm.at[rowv.at[pl.ds(0, CHUNK)]], gbuf0, gsem0).wait()
            _process(gbuf0, j0, cb)

            @pl.when(q < IB // 2 - 1)
            def _():
                pltpu.async_copy(
                    x_hbm.at[rowv.at[pl.ds((j0 + 2) * CHUNK, CHUNK)]],
                    gbuf0, gsem0)
            pltpu.make_async_copy(
                x_hbm.at[rowv.at[pl.ds(0, CHUNK)]], gbuf1, gsem1).wait()
            _process(gbuf1, j1, cb)

            @pl.when(q < IB // 2 - 1)
            def _():
                pltpu.async_copy(
                    x_hbm.at[rowv.at[pl.ds((j1 + 2) * CHUNK, CHUNK)]],
                    gbuf1, gsem1)
            return 0
        lax.fori_loop(0, IB // 2, _pair, 0)
        return 0
    lax.fori_loop(0, nb, _block, 0)

    pltpu.sync_copy(accum, out_hbm.at[pl.ds(colbase, NPW)])


_prop_call = pl.kernel(
    _prop_body,
    out_type=jax.ShapeDtypeStruct((NPAD, HGC), jnp.float32),
    mesh=_mesh,
    scratch_types=[
        pltpu.VMEM((IB * CHUNK,), jnp.int32),
        pltpu.VMEM((IB * CHUNK,), jnp.int32),
        pltpu.VMEM((IB * CHUNK,), jnp.float32),
        pltpu.VMEM((CHUNK, HGC), jnp.float32),
        pltpu.VMEM((CHUNK, HGC), jnp.float32),
        pltpu.VMEM((NPW, HGC), jnp.float32),
        pltpu.VMEM((16,), jnp.int32),
        pltpu.SemaphoreType.DMA,
        pltpu.SemaphoreType.DMA,
    ],
)


def _deg_body(row_hbm, col_hbm, ew_hbm, out_hbm, rowv, colv, ewv, wbuf, deg_sh):
    cid = lax.axis_index("c")
    sid = lax.axis_index("s")

    zero = jnp.zeros((16,), jnp.float32)
    for k in range(8):
        wbuf[pl.ds(k * 16, 16)] = zero
    for q in range(5):
        pltpu.sync_copy(wbuf, deg_sh.at[pl.ds(sid * 640 + q * 128, 128)])

    base = cid * (EP // NC // CHUNK) + sid * CPW
    pltpu.sync_copy(row_hbm.at[pl.ds(base, CPW)], rowv)
    pltpu.sync_copy(col_hbm.at[pl.ds(base, CPW)], colv)
    pltpu.sync_copy(ew_hbm.at[pl.ds(base, CPW)], ewv)

    plsc.subcore_barrier()

    def _chunk(j, _):
        for g in range(8):
            sl = pl.ds(g * 16, 16)
            rv = rowv[j, sl]
            cv = colv[j, sl]
            ev = ewv[j, sl]
            wbuf[sl] = jnp.where(rv != cv, ev, 0.0)
        pltpu.sync_copy(wbuf, deg_sh.at[rowv.at[j]], add=True)
        return 0
    lax.fori_loop(0, CPW, _chunk, 0)

    plsc.subcore_barrier()

    for q in range(5):
        off = sid * 640 + q * 128
        pltpu.sync_copy(deg_sh.at[pl.ds(off, 128)],
                        out_hbm.at[cid, pl.ds(off, 128)])


_deg_call = pl.kernel(
    _deg_body,
    out_type=jax.ShapeDtypeStruct((NC, NPAD), jnp.float32),
    mesh=_mesh,
    scratch_types=[
        pltpu.VMEM((CPW, CHUNK), jnp.int32),
        pltpu.VMEM((CPW, CHUNK), jnp.int32),
        pltpu.VMEM((CPW, CHUNK), jnp.float32),
        pltpu.VMEM((CHUNK,), jnp.float32),
        pltpu.VMEM_SHARED((NPAD,), jnp.float32),
    ],
)


def _dis_body(degp_ref, out_ref):
    deg = degp_ref[0:1, :] + degp_ref[1:2, :]
    out_ref[...] = jnp.where(deg > 0, jax.lax.rsqrt(jnp.where(deg > 0, deg, 1.0)),
                             0.0)


def _dis(degp):
    return pl.pallas_call(
        _dis_body,
        grid=(1,),
        in_specs=[pl.BlockSpec((NC, NPAD), lambda i: (0, 0))],
        out_specs=pl.BlockSpec((1, NPAD), lambda i: (0, 0)),
        out_shape=jax.ShapeDtypeStruct((1, NPAD), jnp.float32),
    )(degp)[0]


def _sum2_body(p_ref, d_ref, tx1_ref, xs_ref):
    d = d_ref[...]
    tx1 = -d * p_ref[...]
    tx1_ref[...] = tx1
    xs_ref[...] = d * tx1


def _sum2(p, dis2):
    R = 2000
    return pl.pallas_call(
        _sum2_body,
        grid=(N // R,),
        in_specs=[pl.BlockSpec((R, HGC), lambda i: (i, 0)),
                  pl.BlockSpec((R, 1), lambda i: (i, 0))],
        out_specs=(pl.BlockSpec((R, HGC), lambda i: (i, 0)),
                   pl.BlockSpec((R, HGC), lambda i: (i, 0))),
        out_shape=(jax.ShapeDtypeStruct((N, HGC), jnp.float32),
                   jax.ShapeDtypeStruct((N, HGC), jnp.float32)),
    )(p, dis2)


def _rowscale_body(x_ref, d_ref, out_ref):
    out_ref[...] = x_ref[...] * d_ref[...]


def _rowscale(x, dis2):
    R = 2000
    return pl.pallas_call(
        _rowscale_body,
        grid=(N // R,),
        in_specs=[pl.BlockSpec((R, HGC), lambda i: (i, 0)),
                  pl.BlockSpec((R, 1), lambda i: (i, 0))],
        out_specs=pl.BlockSpec((R, HGC), lambda i: (i, 0)),
        out_shape=jax.ShapeDtypeStruct((N, HGC), jnp.float32),
    )(x, dis2)


def _layer_body(x_ref, tx1_ref, p2_ref, d_ref, w0_ref, w1_ref, w2_ref,
                h_ref, hs_ref):
    x = x_ref[...]
    d = d_ref[...]
    tx2 = -2.0 * d * p2_ref[...] - x
    acc = jnp.dot(x, w0_ref[...], preferred_element_type=jnp.float32)
    acc += jnp.dot(tx1_ref[...], w1_ref[...], preferred_element_type=jnp.float32)
    acc += jnp.dot(tx2, w2_ref[...], preferred_element_type=jnp.float32)
    h = jnp.maximum(acc, 0.0)
    h_ref[...] = h
    hs_ref[...] = d * h


def _layer_combine(x, tx1, p2, dis2, ws):
    R = 2000
    d_in = x.shape[1]
    return pl.pallas_call(
        _layer_body,
        grid=(N // R,),
        in_specs=[
            pl.BlockSpec((R, d_in), lambda i: (i, 0)),
            pl.BlockSpec((R, HGC), lambda i: (i, 0)),
            pl.BlockSpec((R, HGC), lambda i: (i, 0)),
            pl.BlockSpec((R, 1), lambda i: (i, 0)),
            pl.BlockSpec((d_in, HGC), lambda i: (0, 0)),
            pl.BlockSpec((HGC, HGC), lambda i: (0, 0)),
            pl.BlockSpec((HGC, HGC), lambda i: (0, 0)),
        ],
        out_specs=(pl.BlockSpec((R, HGC), lambda i: (i, 0)),
                   pl.BlockSpec((R, HGC), lambda i: (i, 0))),
        out_shape=(jax.ShapeDtypeStruct((N, HGC), jnp.float32),
                   jax.ShapeDtypeStruct((N, HGC), jnp.float32)),
    )(x, tx1, p2, dis2, ws[0], ws[1], ws[2])


def _pae_body(x_ref, w1_ref, w2_ref, b1_ref, g1_ref, be1_ref, b2_ref, out_ref):
    s = 1.0 / np.sqrt(1.0 + BN_EPS)

    def parser(x):
        h = jnp.dot(x, w1_ref[...], preferred_element_type=jnp.float32)
        h = jnp.maximum(h + b1_ref[...], 0.0)
        h = h * (g1_ref[...] * s) + be1_ref[...]
        return jnp.dot(h, w2_ref[...], preferred_element_type=jnp.float32) + b2_ref[...]

    x = x_ref[...]
    h1 = parser(x[:, :PAE_IN])
    h2 = parser(x[:, PAE_IN:])
    n1 = jnp.maximum(jnp.sqrt(jnp.sum(h1 * h1, axis=1, keepdims=True)), 1e-8)
    n2 = jnp.maximum(jnp.sqrt(jnp.sum(h2 * h2, axis=1, keepdims=True)), 1e-8)
    cos = jnp.sum(h1 * h2, axis=1, keepdims=True) / (n1 * n2)
    out_ref[...] = (cos + 1.0) * 0.5


def _pae(edgenet_input, p):
    Be = 6400
    out = pl.pallas_call(
        _pae_body,
        grid=(E // Be,),
        in_specs=[
            pl.BlockSpec((Be, EDGENET_DIM), lambda i: (i, 0)),
            pl.BlockSpec((PAE_IN, 128), lambda i: (0, 0)),
            pl.BlockSpec((128, 128), lambda i: (0, 0)),
            pl.BlockSpec((1, 128), lambda i: (0, 0)),
            pl.BlockSpec((1, 128), lambda i: (0, 0)),
            pl.BlockSpec((1, 128), lambda i: (0, 0)),
            pl.BlockSpec((1, 128), lambda i: (0, 0)),
        ],
        out_specs=pl.BlockSpec((Be, 1), lambda i: (i, 0)),
        out_shape=jax.ShapeDtypeStruct((E, 1), jnp.float32),
    )(edgenet_input, p["pae_w1"], p["pae_w2"], p["pae_b1"].reshape(1, -1),
      p["pae_g1"].reshape(1, -1), p["pae_be1"].reshape(1, -1),
      p["pae_b2"].reshape(1, -1))
    return out[:, 0]


def _cls_body(h1_ref, h2_ref, h3_ref, h4_ref, w1_ref, b1_ref, g_ref, be_ref,
              w2_ref, b2_ref, out_ref):
    z = jnp.dot(h1_ref[...], w1_ref[0:128, :], preferred_element_type=jnp.float32)
    z += jnp.dot(h2_ref[...], w1_ref[128:256, :], preferred_element_type=jnp.float32)
    z += jnp.dot(h3_ref[...], w1_ref[256:384, :], preferred_element_type=jnp.float32)
    z += jnp.dot(h4_ref[...], w1_ref[384:512, :], preferred_element_type=jnp.float32)
    z = jnp.maximum(z + b1_ref[...], 0.0)
    z = z * (g_ref[...] / np.sqrt(1.0 + BN_EPS)) + be_ref[...]
    out_ref[...] = jnp.dot(z, w2_ref[...], preferred_element_type=jnp.float32) + b2_ref[...]


def _classifier(hs, p):
    R = 2000
    return pl.pallas_call(
        _cls_body,
        grid=(N // R,),
        in_specs=[pl.BlockSpec((R, HGC), lambda i: (i, 0))] * 4 + [
            pl.BlockSpec((HGC * LG, 256), lambda i: (0, 0)),
            pl.BlockSpec((1, 256), lambda i: (0, 0)),
            pl.BlockSpec((1, 256), lambda i: (0, 0)),
            pl.BlockSpec((1, 256), lambda i: (0, 0)),
            pl.BlockSpec((256, NUM_CLASSES), lambda i: (0, 0)),
            pl.BlockSpec((1, NUM_CLASSES), lambda i: (0, 0)),
        ],
        out_specs=pl.BlockSpec((R, NUM_CLASSES), lambda i: (i, 0)),
        out_shape=jax.ShapeDtypeStruct((N, NUM_CLASSES), jnp.float32),
    )(*hs, p["cls_w1"], p["cls_b1"].reshape(1, -1), p["cls_g"].reshape(1, -1),
      p["cls_be"].reshape(1, -1), p["cls_w2"], p["cls_b2"].reshape(1, -1))


def kernel(features, edge_index, edgenet_input, params):
    p = params
    ew = _pae(edgenet_input, p)

    row, col = edge_index[0], edge_index[1]
    pad = EP - E
    row2d = jnp.concatenate([row, jnp.zeros((pad,), jnp.int32)]).reshape(-1, CHUNK)
    col2d = jnp.concatenate([col, jnp.zeros((pad,), jnp.int32)]).reshape(-1, CHUNK)
    ew2d = jnp.concatenate([ew, jnp.zeros((pad,), jnp.float32)]).reshape(-1, CHUNK)

    degp = _deg_call(row2d, col2d, ew2d)
    dis = _dis(degp)
    dis2 = dis[:N].reshape(N, 1)

    col_p = col2d.reshape(-1)
    perm = jnp.argsort(col_p)
    zpad = jnp.zeros((EP1 - EP,), jnp.int32)
    row_s = jnp.concatenate([row2d.reshape(-1)[perm], zpad])
    col_s = jnp.concatenate([col_p[perm], zpad])
    ew_s = jnp.concatenate([ew2d.reshape(-1)[perm], zpad.astype(jnp.float32)])
    edges = jnp.arange(0, NPAD + 1, NPW, dtype=jnp.int32)
    bnd = jnp.searchsorted(col_s[:EP], edges).astype(jnp.int32)
    bounds = (jnp.zeros((NW, 16), jnp.int32)
              .at[:, 0].set(bnd[:NW]).at[:, 1].set(bnd[1:]).reshape(-1))

    h = features
    hsc = _rowscale(features, dis2)
    hs = []
    for i in range(LG):
        p1 = _prop_call(hsc, row_s, col_s, ew_s, bounds)
        tx1, xs1 = _sum2(p1[:N], dis2)
        p2 = _prop_call(xs1, row_s, col_s, ew_s, bounds)
        h, hsc = _layer_combine(h, tx1, p2[:N], dis2, p["cheb"][i])
        hs.append(h)
    logit = _classifier(hs, p)
    return logit, ew


# speedup vs baseline: 2.3476x; 2.3476x over previous
import functools

import jax
import jax.numpy as jnp
import numpy as np
from jax import lax
from jax.experimental import pallas as pl
from jax.experimental.pallas import tpu as pltpu
from jax.experimental.pallas import tpu_sc as plsc

N = 10000
E = 320000
D_IN = 128
HGC = 128
LG = 4
NUM_CLASSES = 2
EDGENET_DIM = 16
PAE_IN = EDGENET_DIM // 2
BN_EPS = 1e-5

NC = 2
NS = 16
NW = NC * NS
CHUNK = 128
CPW = 80
EP = NW * CPW * CHUNK
EPW = CPW * CHUNK
NPAD = 10240
ZROWS = NPAD // NS // 5

_mesh = plsc.VectorSubcoreMesh(
    core_axis_name="c", subcore_axis_name="s", num_cores=NC, num_subcores=NS)


IB = 16
NBLK = CPW // IB
PB = IB // 2


def _prop_body(x_hbm, row_hbm, col_hbm, ew_hbm, out_hbm,
               rowv, colv, ewv, gbuf0, gbuf1, out_sh,
               gsem0, gsem1, ssem0, ssem1):
    cid = lax.axis_index("c")
    sid = lax.axis_index("s")

    zero = jnp.zeros((16,), jnp.float32)
    def _zrow(i, _):
        for k in range(8):
            gbuf0[i, pl.ds(k * 16, 16)] = zero
        return 0
    lax.fori_loop(0, ZROWS, _zrow, 0)
    for q in range(5):
        pltpu.sync_copy(gbuf0, out_sh.at[pl.ds(sid * (ZROWS * 5) + q * ZROWS,
                                               ZROWS)])

    base = cid * (EP // NC // CHUNK) + sid * CPW
    plsc.subcore_barrier()

    def _scale(buf, j):
        def _group(g, _):
            sl = pl.ds(g * 16, 16)
            sv = jnp.where(rowv[j, sl] != colv[j, sl], ewv[j, sl], 0.0)
            for e in range(16):
                s = sv[e]
                for k in range(8):
                    buf[g * 16 + e, pl.ds(k * 16, 16)] = (
                        buf[g * 16 + e, pl.ds(k * 16, 16)] * s)
            return 0
        lax.fori_loop(0, CHUNK // 16, _group, 0)

    for blk in range(NBLK):
        pltpu.sync_copy(row_hbm.at[pl.ds(base + blk * IB, IB)], rowv)
        pltpu.sync_copy(col_hbm.at[pl.ds(base + blk * IB, IB)], colv)
        pltpu.sync_copy(ew_hbm.at[pl.ds(base + blk * IB, IB)], ewv)
        pltpu.async_copy(x_hbm.at[rowv.at[0]], gbuf0, gsem0)
        pltpu.async_copy(x_hbm.at[rowv.at[1]], gbuf1, gsem1)

        def _pair(q, _):
            j0 = 2 * q
            j1 = j0 + 1
            pltpu.make_async_copy(x_hbm.at[rowv.at[j0]], gbuf0, gsem0).wait()
            _scale(gbuf0, j0)
            pltpu.async_copy(gbuf0, out_sh.at[colv.at[j0]], ssem0, add=True)
            pltpu.make_async_copy(x_hbm.at[rowv.at[j1]], gbuf1, gsem1).wait()
            _scale(gbuf1, j1)
            pltpu.async_copy(gbuf1, out_sh.at[colv.at[j1]], ssem1, add=True)

            @pl.when(q < PB - 1)
            def _():
                pltpu.make_async_copy(gbuf0, out_sh.at[colv.at[j0]], ssem0).wait()
                pltpu.async_copy(x_hbm.at[rowv.at[j0 + 2]], gbuf0, gsem0)
                pltpu.make_async_copy(gbuf1, out_sh.at[colv.at[j1]], ssem1).wait()
                pltpu.async_copy(x_hbm.at[rowv.at[j1 + 2]], gbuf1, gsem1)

            @pl.when(q == PB - 1)
            def _():
                pltpu.make_async_copy(gbuf0, out_sh.at[colv.at[j0]], ssem0).wait()
                pltpu.make_async_copy(gbuf1, out_sh.at[colv.at[j1]], ssem1).wait()
            return 0
        lax.fori_loop(0, PB, _pair, 0)

    plsc.subcore_barrier()

    for q in range(5):
        off = sid * (ZROWS * 5) + q * ZROWS
        pltpu.sync_copy(out_sh.at[pl.ds(off, ZROWS)],
                        out_hbm.at[cid, pl.ds(off, ZROWS)])


def _deg_body(row_hbm, col_hbm, ew_hbm, out_hbm, rowv, colv, ewv, wbuf, deg_sh):
    cid = lax.axis_index("c")
    sid = lax.axis_index("s")

    zero = jnp.zeros((16,), jnp.float32)
    for k in range(8):
        wbuf[pl.ds(k * 16, 16)] = zero
    for q in range(5):
        pltpu.sync_copy(wbuf, deg_sh.at[pl.ds(sid * 640 + q * 128, 128)])

    base = cid * (EP // NC // CHUNK) + sid * CPW
    pltpu.sync_copy(row_hbm.at[pl.ds(base, CPW)], rowv)
    pltpu.sync_copy(col_hbm.at[pl.ds(base, CPW)], colv)
    pltpu.sync_copy(ew_hbm.at[pl.ds(base, CPW)], ewv)

    plsc.subcore_barrier()

    def _chunk(j, _):
        for g in range(8):
            sl = pl.ds(g * 16, 16)
            rv = rowv[j, sl]
            cv = colv[j, sl]
            ev = ewv[j, sl]
            wbuf[sl] = jnp.where(rv != cv, ev, 0.0)
        pltpu.sync_copy(wbuf, deg_sh.at[rowv.at[j]], add=True)
        return 0
    lax.fori_loop(0, CPW, _chunk, 0)

    plsc.subcore_barrier()

    for q in range(5):
        off = sid * 640 + q * 128
        pltpu.sync_copy(deg_sh.at[pl.ds(off, 128)],
                        out_hbm.at[cid, pl.ds(off, 128)])


_deg_call = pl.kernel(
    _deg_body,
    out_type=jax.ShapeDtypeStruct((NC, NPAD), jnp.float32),
    mesh=_mesh,
    scratch_types=[
        pltpu.VMEM((CPW, CHUNK), jnp.int32),
        pltpu.VMEM((CPW, CHUNK), jnp.int32),
        pltpu.VMEM((CPW, CHUNK), jnp.float32),
        pltpu.VMEM((CHUNK,), jnp.float32),
        pltpu.VMEM_SHARED((NPAD,), jnp.float32),
    ],
)


def _dis_body(degp_ref, out_ref):
    deg = degp_ref[0:1, :] + degp_ref[1:2, :]
    out_ref[...] = jnp.where(deg > 0, jax.lax.rsqrt(jnp.where(deg > 0, deg, 1.0)),
                             0.0)


def _dis(degp):
    return pl.pallas_call(
        _dis_body,
        grid=(1,),
        in_specs=[pl.BlockSpec((NC, NPAD), lambda i: (0, 0))],
        out_specs=pl.BlockSpec((1, NPAD), lambda i: (0, 0)),
        out_shape=jax.ShapeDtypeStruct((1, NPAD), jnp.float32),
    )(degp)[0]


_prop_call = pl.kernel(
    _prop_body,
    out_type=jax.ShapeDtypeStruct((NC, NPAD, HGC), jnp.float32),
    mesh=_mesh,
    scratch_types=[
        pltpu.VMEM((IB, CHUNK), jnp.int32),
        pltpu.VMEM((IB, CHUNK), jnp.int32),
        pltpu.VMEM((IB, CHUNK), jnp.float32),
        pltpu.VMEM((CHUNK, HGC), jnp.float32),
        pltpu.VMEM((CHUNK, HGC), jnp.float32),
        pltpu.VMEM_SHARED((NPAD, HGC), jnp.float32),
        pltpu.SemaphoreType.DMA,
        pltpu.SemaphoreType.DMA,
        pltpu.SemaphoreType.DMA,
        pltpu.SemaphoreType.DMA,
    ],
)


def _sum2_body(p_ref, d_ref, tx1_ref, xs_ref):
    d = d_ref[...]
    tx1 = -d * (p_ref[0] + p_ref[1])
    tx1_ref[...] = tx1
    xs_ref[...] = d * tx1


def _sum2(p, dis2):
    R = 2000
    return pl.pallas_call(
        _sum2_body,
        grid=(N // R,),
        in_specs=[pl.BlockSpec((NC, R, HGC), lambda i: (0, i, 0)),
                  pl.BlockSpec((R, 1), lambda i: (i, 0))],
        out_specs=(pl.BlockSpec((R, HGC), lambda i: (i, 0)),
                   pl.BlockSpec((R, HGC), lambda i: (i, 0))),
        out_shape=(jax.ShapeDtypeStruct((N, HGC), jnp.float32),
                   jax.ShapeDtypeStruct((N, HGC), jnp.float32)),
    )(p, dis2)


def _rowscale_body(x_ref, d_ref, out_ref):
    out_ref[...] = x_ref[...] * d_ref[...]


def _rowscale(x, dis2):
    R = 2000
    return pl.pallas_call(
        _rowscale_body,
        grid=(N // R,),
        in_specs=[pl.BlockSpec((R, HGC), lambda i: (i, 0)),
                  pl.BlockSpec((R, 1), lambda i: (i, 0))],
        out_specs=pl.BlockSpec((R, HGC), lambda i: (i, 0)),
        out_shape=jax.ShapeDtypeStruct((N, HGC), jnp.float32),
    )(x, dis2)


def _layer_body(x_ref, tx1_ref, p2_ref, d_ref, w0_ref, w1_ref, w2_ref,
                h_ref, hs_ref):
    x = x_ref[...]
    d = d_ref[...]
    tx2 = -2.0 * d * (p2_ref[0] + p2_ref[1]) - x
    acc = jnp.dot(x, w0_ref[...], preferred_element_type=jnp.float32)
    acc += jnp.dot(tx1_ref[...], w1_ref[...], preferred_element_type=jnp.float32)
    acc += jnp.dot(tx2, w2_ref[...], preferred_element_type=jnp.float32)
    h = jnp.maximum(acc, 0.0)
    h_ref[...] = h
    hs_ref[...] = d * h


def _layer_combine(x, tx1, p2, dis2, ws):
    R = 2000
    d_in = x.shape[1]
    return pl.pallas_call(
        _layer_body,
        grid=(N // R,),
        in_specs=[
            pl.BlockSpec((R, d_in), lambda i: (i, 0)),
            pl.BlockSpec((R, HGC), lambda i: (i, 0)),
            pl.BlockSpec((NC, R, HGC), lambda i: (0, i, 0)),
            pl.BlockSpec((R, 1), lambda i: (i, 0)),
            pl.BlockSpec((d_in, HGC), lambda i: (0, 0)),
            pl.BlockSpec((HGC, HGC), lambda i: (0, 0)),
            pl.BlockSpec((HGC, HGC), lambda i: (0, 0)),
        ],
        out_specs=(pl.BlockSpec((R, HGC), lambda i: (i, 0)),
                   pl.BlockSpec((R, HGC), lambda i: (i, 0))),
        out_shape=(jax.ShapeDtypeStruct((N, HGC), jnp.float32),
                   jax.ShapeDtypeStruct((N, HGC), jnp.float32)),
    )(x, tx1, p2, dis2, ws[0], ws[1], ws[2])


def _pae_body(x_ref, w1_ref, w2_ref, b1_ref, g1_ref, be1_ref, b2_ref, out_ref):
    s = 1.0 / np.sqrt(1.0 + BN_EPS)

    def parser(x):
        h = jnp.dot(x, w1_ref[...], preferred_element_type=jnp.float32)
        h = jnp.maximum(h + b1_ref[...], 0.0)
        h = h * (g1_ref[...] * s) + be1_ref[...]
        return jnp.dot(h, w2_ref[...], preferred_element_type=jnp.float32) + b2_ref[...]

    x = x_ref[...]
    h1 = parser(x[:, :PAE_IN])
    h2 = parser(x[:, PAE_IN:])
    n1 = jnp.maximum(jnp.sqrt(jnp.sum(h1 * h1, axis=1, keepdims=True)), 1e-8)
    n2 = jnp.maximum(jnp.sqrt(jnp.sum(h2 * h2, axis=1, keepdims=True)), 1e-8)
    cos = jnp.sum(h1 * h2, axis=1, keepdims=True) / (n1 * n2)
    out_ref[...] = (cos + 1.0) * 0.5


def _pae(edgenet_input, p):
    Be = 6400
    out = pl.pallas_call(
        _pae_body,
        grid=(E // Be,),
        in_specs=[
            pl.BlockSpec((Be, EDGENET_DIM), lambda i: (i, 0)),
            pl.BlockSpec((PAE_IN, 128), lambda i: (0, 0)),
            pl.BlockSpec((128, 128), lambda i: (0, 0)),
            pl.BlockSpec((1, 128), lambda i: (0, 0)),
            pl.BlockSpec((1, 128), lambda i: (0, 0)),
            pl.BlockSpec((1, 128), lambda i: (0, 0)),
            pl.BlockSpec((1, 128), lambda i: (0, 0)),
        ],
        out_specs=pl.BlockSpec((Be, 1), lambda i: (i, 0)),
        out_shape=jax.ShapeDtypeStruct((E, 1), jnp.float32),
    )(edgenet_input, p["pae_w1"], p["pae_w2"], p["pae_b1"].reshape(1, -1),
      p["pae_g1"].reshape(1, -1), p["pae_be1"].reshape(1, -1),
      p["pae_b2"].reshape(1, -1))
    return out[:, 0]


def _cls_body(h1_ref, h2_ref, h3_ref, h4_ref, w1_ref, b1_ref, g_ref, be_ref,
              w2_ref, b2_ref, out_ref):
    z = jnp.dot(h1_ref[...], w1_ref[0:128, :], preferred_element_type=jnp.float32)
    z += jnp.dot(h2_ref[...], w1_ref[128:256, :], preferred_element_type=jnp.float32)
    z += jnp.dot(h3_ref[...], w1_ref[256:384, :], preferred_element_type=jnp.float32)
    z += jnp.dot(h4_ref[...], w1_ref[384:512, :], preferred_element_type=jnp.float32)
    z = jnp.maximum(z + b1_ref[...], 0.0)
    z = z * (g_ref[...] / np.sqrt(1.0 + BN_EPS)) + be_ref[...]
    out_ref[...] = jnp.dot(z, w2_ref[...], preferred_element_type=jnp.float32) + b2_ref[...]


def _classifier(hs, p):
    R = 2000
    return pl.pallas_call(
        _cls_body,
        grid=(N // R,),
        in_specs=[pl.BlockSpec((R, HGC), lambda i: (i, 0))] * 4 + [
            pl.BlockSpec((HGC * LG, 256), lambda i: (0, 0)),
            pl.BlockSpec((1, 256), lambda i: (0, 0)),
            pl.BlockSpec((1, 256), lambda i: (0, 0)),
            pl.BlockSpec((1, 256), lambda i: (0, 0)),
            pl.BlockSpec((256, NUM_CLASSES), lambda i: (0, 0)),
            pl.BlockSpec((1, NUM_CLASSES), lambda i: (0, 0)),
        ],
        out_specs=pl.BlockSpec((R, NUM_CLASSES), lambda i: (i, 0)),
        out_shape=jax.ShapeDtypeStruct((N, NUM_CLASSES), jnp.float32),
    )(*hs, p["cls_w1"], p["cls_b1"].reshape(1, -1), p["cls_g"].reshape(1, -1),
      p["cls_be"].reshape(1, -1), p["cls_w2"], p["cls_b2"].reshape(1, -1))


def kernel(features, edge_index, edgenet_input, params):
    p = params
    ew = _pae(edgenet_input, p)

    row, col = edge_index[0], edge_index[1]
    pad = EP - E
    row2d = jnp.concatenate([row, jnp.zeros((pad,), jnp.int32)]).reshape(-1, CHUNK)
    col2d = jnp.concatenate([col, jnp.zeros((pad,), jnp.int32)]).reshape(-1, CHUNK)
    ew2d = jnp.concatenate([ew, jnp.zeros((pad,), jnp.float32)]).reshape(-1, CHUNK)

    degp = _deg_call(row2d, col2d, ew2d)
    dis = _dis(degp)
    dis2 = dis[:N].reshape(N, 1)

    h = features
    hsc = _rowscale(features, dis2)
    hs = []
    for i in range(LG):
        p1 = _prop_call(hsc, row2d, col2d, ew2d)
        tx1, xs1 = _sum2(p1, dis2)
        p2 = _prop_call(xs1, row2d, col2d, ew2d)
        h, hsc = _layer_combine(h, tx1, p2, dis2, p["cheb"][i])
        hs.append(h)
    logit = _classifier(hs, p)
    return logit, ew


# speedup vs baseline: 2.3534x; 1.0024x over previous
import functools

import jax
import jax.numpy as jnp
import numpy as np
from jax import lax
from jax.experimental import pallas as pl
from jax.experimental.pallas import tpu as pltpu
from jax.experimental.pallas import tpu_sc as plsc

N = 10000
E = 320000
D_IN = 128
HGC = 128
LG = 4
NUM_CLASSES = 2
EDGENET_DIM = 16
PAE_IN = EDGENET_DIM // 2
BN_EPS = 1e-5

NC = 2
NS = 16
NW = NC * NS
CHUNK = 128
CPW = 80
EP = NW * CPW * CHUNK
EPW = CPW * CHUNK
NPAD = 10240
ZROWS = NPAD // NS // 5

IB = 16
NBLK = CPW // IB
PB = IB // 2


def _prop_body(x_hbm, row_hbm, col_hbm, ew_hbm, out_hbm,
               rowv, colv, ewv, gbuf0, gbuf1, out_sh,
               gsem0, gsem1, ssem0, ssem1):
    cid = lax.axis_index("c")
    sid = lax.axis_index("s")

    zero = jnp.zeros((16,), jnp.float32)
    def _zrow(i, _):
        for k in range(8):
            gbuf0[i, pl.ds(k * 16, 16)] = zero
        return 0
    lax.fori_loop(0, ZROWS, _zrow, 0)
    for q in range(5):
        pltpu.sync_copy(gbuf0, out_sh.at[pl.ds(sid * (ZROWS * 5) + q * ZROWS,
                                               ZROWS)])

    base = cid * (EP // NC // CHUNK) + sid * CPW
    plsc.subcore_barrier()

    def _scale(buf, j):
        def _group(g, _):
            sl = pl.ds(g * 16, 16)
            sv = jnp.where(rowv[j, sl] != colv[j, sl], ewv[j, sl], 0.0)
            for e in range(16):
                s = sv[e]
                for k in range(8):
                    buf[g * 16 + e, pl.ds(k * 16, 16)] = (
                        buf[g * 16 + e, pl.ds(k * 16, 16)] * s)
            return 0
        lax.fori_loop(0, CHUNK // 16, _group, 0)

    for blk in range(NBLK):
        pltpu.sync_copy(row_hbm.at[pl.ds(base + blk * IB, IB)], rowv)
        pltpu.sync_copy(col_hbm.at[pl.ds(base + blk * IB, IB)], colv)
        pltpu.sync_copy(ew_hbm.at[pl.ds(base + blk * IB, IB)], ewv)
        pltpu.async_copy(x_hbm.at[rowv.at[0]], gbuf0, gsem0)
        pltpu.async_copy(x_hbm.at[rowv.at[1]], gbuf1, gsem1)

        def _pair(q, _):
            j0 = 2 * q
            j1 = j0 + 1
            pltpu.make_async_copy(x_hbm.at[rowv.at[j0]], gbuf0, gsem0).wait()
            _scale(gbuf0, j0)
            pltpu.async_copy(gbuf0, out_sh.at[colv.at[j0]], ssem0, add=True)
            pltpu.make_async_copy(x_hbm.at[rowv.at[j1]], gbuf1, gsem1).wait()
            _scale(gbuf1, j1)
            pltpu.async_copy(gbuf1, out_sh.at[colv.at[j1]], ssem1, add=True)

            @pl.when(q < PB - 1)
            def _():
                pltpu.make_async_copy(gbuf0, out_sh.at[colv.at[j0]], ssem0).wait()
                pltpu.async_copy(x_hbm.at[rowv.at[j0 + 2]], gbuf0, gsem0)
                pltpu.make_async_copy(gbuf1, out_sh.at[colv.at[j1]], ssem1).wait()
                pltpu.async_copy(x_hbm.at[rowv.at[j1 + 2]], gbuf1, gsem1)

            @pl.when(q == PB - 1)
            def _():
                pltpu.make_async_copy(gbuf0, out_sh.at[colv.at[j0]], ssem0).wait()
                pltpu.make_async_copy(gbuf1, out_sh.at[colv.at[j1]], ssem1).wait()
            return 0
        lax.fori_loop(0, PB, _pair, 0)

    plsc.subcore_barrier()

    for q in range(5):
        off = sid * (ZROWS * 5) + q * ZROWS
        pltpu.sync_copy(out_sh.at[pl.ds(off, ZROWS)],
                        out_hbm.at[cid, pl.ds(off, ZROWS)])


def _deg_body(row_hbm, col_hbm, ew_hbm, out_hbm, rowv, colv, ewv, wbuf, deg_sh):
    cid = lax.axis_index("c")
    sid = lax.axis_index("s")

    zero = jnp.zeros((16,), jnp.float32)
    for k in range(8):
        wbuf[pl.ds(k * 16, 16)] = zero
    for q in range(5):
        pltpu.sync_copy(wbuf, deg_sh.at[pl.ds(sid * 640 + q * 128, 128)])

    base = cid * (EP // NC // CHUNK) + sid * CPW
    pltpu.sync_copy(row_hbm.at[pl.ds(base, CPW)], rowv)
    pltpu.sync_copy(col_hbm.at[pl.ds(base, CPW)], colv)
    pltpu.sync_copy(ew_hbm.at[pl.ds(base, CPW)], ewv)

    plsc.subcore_barrier()

    def _chunk(j, _):
        for g in range(8):
            sl = pl.ds(g * 16, 16)
            rv = rowv[j, sl]
            cv = colv[j, sl]
            ev = ewv[j, sl]
            wbuf[sl] = jnp.where(rv != cv, ev, 0.0)
        pltpu.sync_copy(wbuf, deg_sh.at[rowv.at[j]], add=True)
        return 0
    lax.fori_loop(0, CPW, _chunk, 0)

    plsc.subcore_barrier()

    for q in range(5):
        off = sid * 640 + q * 128
        pltpu.sync_copy(deg_sh.at[pl.ds(off, 128)],
                        out_hbm.at[cid, pl.ds(off, 128)])


def _dis_body(degp_ref, out_ref):
    deg = degp_ref[0:1, :] + degp_ref[1:2, :]
    out_ref[...] = jnp.where(deg > 0, jax.lax.rsqrt(jnp.where(deg > 0, deg, 1.0)),
                             0.0)


def _dis(degp):
    return pl.pallas_call(
        _dis_body,
        grid=(1,),
        in_specs=[pl.BlockSpec((NC, NPAD), lambda i: (0, 0))],
        out_specs=pl.BlockSpec((1, NPAD), lambda i: (0, 0)),
        out_shape=jax.ShapeDtypeStruct((1, NPAD), jnp.float32),
    )(degp)[0]


@functools.cache
def _sc_kernels():
    mesh = plsc.VectorSubcoreMesh(
        core_axis_name="c", subcore_axis_name="s",
        num_cores=NC, num_subcores=NS)
    prop = pl.kernel(
        _prop_body,
        out_type=jax.ShapeDtypeStruct((NC, NPAD, HGC), jnp.float32),
        mesh=mesh,
        scratch_types=[
            pltpu.VMEM((IB, CHUNK), jnp.int32),
            pltpu.VMEM((IB, CHUNK), jnp.int32),
            pltpu.VMEM((IB, CHUNK), jnp.float32),
            pltpu.VMEM((CHUNK, HGC), jnp.float32),
            pltpu.VMEM((CHUNK, HGC), jnp.float32),
            pltpu.VMEM_SHARED((NPAD, HGC), jnp.float32),
            pltpu.SemaphoreType.DMA,
            pltpu.SemaphoreType.DMA,
            pltpu.SemaphoreType.DMA,
            pltpu.SemaphoreType.DMA,
        ],
    )
    deg = pl.kernel(
        _deg_body,
        out_type=jax.ShapeDtypeStruct((NC, NPAD), jnp.float32),
        mesh=mesh,
        scratch_types=[
            pltpu.VMEM((CPW, CHUNK), jnp.int32),
            pltpu.VMEM((CPW, CHUNK), jnp.int32),
            pltpu.VMEM((CPW, CHUNK), jnp.float32),
            pltpu.VMEM((CHUNK,), jnp.float32),
            pltpu.VMEM_SHARED((NPAD,), jnp.float32),
        ],
    )
    return prop, deg


def _sum2_body(p_ref, d_ref, tx1_ref, xs_ref):
    d = d_ref[...]
    tx1 = -d * (p_ref[0] + p_ref[1])
    tx1_ref[...] = tx1
    xs_ref[...] = d * tx1


def _sum2(p, dis2):
    R = 2000
    return pl.pallas_call(
        _sum2_body,
        grid=(N // R,),
        in_specs=[pl.BlockSpec((NC, R, HGC), lambda i: (0, i, 0)),
                  pl.BlockSpec((R, 1), lambda i: (i, 0))],
        out_specs=(pl.BlockSpec((R, HGC), lambda i: (i, 0)),
                   pl.BlockSpec((R, HGC), lambda i: (i, 0))),
        out_shape=(jax.ShapeDtypeStruct((N, HGC), jnp.float32),
                   jax.ShapeDtypeStruct((N, HGC), jnp.float32)),
    )(p, dis2)


def _rowscale_body(x_ref, d_ref, out_ref):
    out_ref[...] = x_ref[...] * d_ref[...]


def _rowscale(x, dis2):
    R = 2000
    return pl.pallas_call(
        _rowscale_body,
        grid=(N // R,),
        in_specs=[pl.BlockSpec((R, HGC), lambda i: (i, 0)),
                  pl.BlockSpec((R, 1), lambda i: (i, 0))],
        out_specs=pl.BlockSpec((R, HGC), lambda i: (i, 0)),
        out_shape=jax.ShapeDtypeStruct((N, HGC), jnp.float32),
    )(x, dis2)


def _layer_body(x_ref, tx1_ref, p2_ref, d_ref, w0_ref, w1_ref, w2_ref,
                h_ref, hs_ref):
    x = x_ref[...]
    d = d_ref[...]
    tx2 = -2.0 * d * (p2_ref[0] + p2_ref[1]) - x
    acc = jnp.dot(x, w0_ref[...], preferred_element_type=jnp.float32)
    acc += jnp.dot(tx1_ref[...], w1_ref[...], preferred_element_type=jnp.float32)
    acc += jnp.dot(tx2, w2_ref[...], preferred_element_type=jnp.float32)
    h = jnp.maximum(acc, 0.0)
    h_ref[...] = h
    hs_ref[...] = d * h


def _layer_combine(x, tx1, p2, dis2, ws):
    R = 2000
    d_in = x.shape[1]
    return pl.pallas_call(
        _layer_body,
        grid=(N // R,),
        in_specs=[
            pl.BlockSpec((R, d_in), lambda i: (i, 0)),
            pl.BlockSpec((R, HGC), lambda i: (i, 0)),
            pl.BlockSpec((NC, R, HGC), lambda i: (0, i, 0)),
            pl.BlockSpec((R, 1), lambda i: (i, 0)),
            pl.BlockSpec((d_in, HGC), lambda i: (0, 0)),
            pl.BlockSpec((HGC, HGC), lambda i: (0, 0)),
            pl.BlockSpec((HGC, HGC), lambda i: (0, 0)),
        ],
        out_specs=(pl.BlockSpec((R, HGC), lambda i: (i, 0)),
                   pl.BlockSpec((R, HGC), lambda i: (i, 0))),
        out_shape=(jax.ShapeDtypeStruct((N, HGC), jnp.float32),
                   jax.ShapeDtypeStruct((N, HGC), jnp.float32)),
    )(x, tx1, p2, dis2, ws[0], ws[1], ws[2])


def _pae_body(x_ref, w1_ref, w2_ref, b1_ref, g1_ref, be1_ref, b2_ref, out_ref):
    s = 1.0 / np.sqrt(1.0 + BN_EPS)

    def parser(x):
        h = jnp.dot(x, w1_ref[...], preferred_element_type=jnp.float32)
        h = jnp.maximum(h + b1_ref[...], 0.0)
        h = h * (g1_ref[...] * s) + be1_ref[...]
        return jnp.dot(h, w2_ref[...], preferred_element_type=jnp.float32) + b2_ref[...]

    x = x_ref[...]
    h1 = parser(x[:, :PAE_IN])
    h2 = parser(x[:, PAE_IN:])
    n1 = jnp.maximum(jnp.sqrt(jnp.sum(h1 * h1, axis=1, keepdims=True)), 1e-8)
    n2 = jnp.maximum(jnp.sqrt(jnp.sum(h2 * h2, axis=1, keepdims=True)), 1e-8)
    cos = jnp.sum(h1 * h2, axis=1, keepdims=True) / (n1 * n2)
    out_ref[...] = (cos + 1.0) * 0.5


def _pae(edgenet_input, p):
    Be = 6400
    out = pl.pallas_call(
        _pae_body,
        grid=(E // Be,),
        in_specs=[
            pl.BlockSpec((Be, EDGENET_DIM), lambda i: (i, 0)),
            pl.BlockSpec((PAE_IN, 128), lambda i: (0, 0)),
            pl.BlockSpec((128, 128), lambda i: (0, 0)),
            pl.BlockSpec((1, 128), lambda i: (0, 0)),
            pl.BlockSpec((1, 128), lambda i: (0, 0)),
            pl.BlockSpec((1, 128), lambda i: (0, 0)),
            pl.BlockSpec((1, 128), lambda i: (0, 0)),
        ],
        out_specs=pl.BlockSpec((Be, 1), lambda i: (i, 0)),
        out_shape=jax.ShapeDtypeStruct((E, 1), jnp.float32),
    )(edgenet_input, p["pae_w1"], p["pae_w2"], p["pae_b1"].reshape(1, -1),
      p["pae_g1"].reshape(1, -1), p["pae_be1"].reshape(1, -1),
      p["pae_b2"].reshape(1, -1))
    return out[:, 0]


def _cls_body(h1_ref, h2_ref, h3_ref, h4_ref, w1_ref, b1_ref, g_ref, be_ref,
              w2_ref, b2_ref, out_ref):
    z = jnp.dot(h1_ref[...], w1_ref[0:128, :], preferred_element_type=jnp.float32)
    z += jnp.dot(h2_ref[...], w1_ref[128:256, :], preferred_element_type=jnp.float32)
    z += jnp.dot(h3_ref[...], w1_ref[256:384, :], preferred_element_type=jnp.float32)
    z += jnp.dot(h4_ref[...], w1_ref[384:512, :], preferred_element_type=jnp.float32)
    z = jnp.maximum(z + b1_ref[...], 0.0)
    z = z * (g_ref[...] / np.sqrt(1.0 + BN_EPS)) + be_ref[...]
    out_ref[...] = jnp.dot(z, w2_ref[...], preferred_element_type=jnp.float32) + b2_ref[...]


def _classifier(hs, p):
    R = 2000
    return pl.pallas_call(
        _cls_body,
        grid=(N // R,),
        in_specs=[pl.BlockSpec((R, HGC), lambda i: (i, 0))] * 4 + [
            pl.BlockSpec((HGC * LG, 256), lambda i: (0, 0)),
            pl.BlockSpec((1, 256), lambda i: (0, 0)),
            pl.BlockSpec((1, 256), lambda i: (0, 0)),
            pl.BlockSpec((1, 256), lambda i: (0, 0)),
            pl.BlockSpec((256, NUM_CLASSES), lambda i: (0, 0)),
            pl.BlockSpec((1, NUM_CLASSES), lambda i: (0, 0)),
        ],
        out_specs=pl.BlockSpec((R, NUM_CLASSES), lambda i: (i, 0)),
        out_shape=jax.ShapeDtypeStruct((N, NUM_CLASSES), jnp.float32),
    )(*hs, p["cls_w1"], p["cls_b1"].reshape(1, -1), p["cls_g"].reshape(1, -1),
      p["cls_be"].reshape(1, -1), p["cls_w2"], p["cls_b2"].reshape(1, -1))


def kernel(features, edge_index, edgenet_input, params):
    p = params
    ew = _pae(edgenet_input, p)

    row, col = edge_index[0], edge_index[1]
    pad = EP - E
    row2d = jnp.concatenate([row, jnp.zeros((pad,), jnp.int32)]).reshape(-1, CHUNK)
    col2d = jnp.concatenate([col, jnp.zeros((pad,), jnp.int32)]).reshape(-1, CHUNK)
    ew2d = jnp.concatenate([ew, jnp.zeros((pad,), jnp.float32)]).reshape(-1, CHUNK)

    _prop_call, _deg_call = _sc_kernels()
    degp = _deg_call(row2d, col2d, ew2d)
    dis = _dis(degp)
    dis2 = dis[:N].reshape(N, 1)

    h = features
    hsc = _rowscale(features, dis2)
    hs = []
    for i in range(LG):
        p1 = _prop_call(hsc, row2d, col2d, ew2d)
        tx1, xs1 = _sum2(p1, dis2)
        p2 = _prop_call(xs1, row2d, col2d, ew2d)
        h, hsc = _layer_combine(h, tx1, p2, dis2, p["cheb"][i])
        hs.append(h)
    logit = _classifier(hs, p)
    return logit, ew
